# Initial kernel scaffold; baseline (speedup 1.0000x reference)
import jax
import jax.numpy as jnp
from jax.experimental import pallas as pl


def kernel(z, edge_index, W1l, W1r, b1, W2l, W2r, b2, W3l, W3r, b3, W4l, W4r, b4, g1, be1, g2, be2, g3, be3):
    raise NotImplementedError("write your pallas kernel here")


# speedup vs baseline: 4.3005x; 4.3005x over previous
import functools

import jax
import jax.numpy as jnp
from jax import lax
from jax.experimental import pallas as pl
from jax.experimental.pallas import tpu as pltpu, tpu_sc as plsc

_NC = 2
_NS = 16
_K = 80


def _segsum_sc(table, src, dst, with_cnt):
    N, D = table.shape
    E = src.shape[0]
    NW = _NC * _NS
    epw = E // NW
    nblk = epw // _K
    assert epw * NW == E and nblk * _K == epw
    rpt = (N // _NS) & ~7
    rem = N - _NS * rpt

    CH = 48
    assert rpt % CH == 0 and rem <= CH

    mesh = plsc.VectorSubcoreMesh(core_axis_name="c", subcore_axis_name="s")
    out_type = [jax.ShapeDtypeStruct((_NC, N, D), jnp.float32)]
    scratch = [
        pltpu.VMEM_SHARED((N, D), jnp.float32),
        pltpu.VMEM((_K,), jnp.int32),
        pltpu.VMEM((_K,), jnp.int32),
        pltpu.VMEM((_K, D), jnp.float32),
        pltpu.VMEM((CH, D), jnp.float32),
        pltpu.SemaphoreType.DMA,
    ]
    if with_cnt:
        out_type.append(jax.ShapeDtypeStruct((_NC, N, 16), jnp.float32))
        scratch += [
            pltpu.VMEM_SHARED((N, 16), jnp.float32),
            pltpu.VMEM((_K, 16), jnp.float32),
            pltpu.VMEM((CH, 16), jnp.float32),
        ]

    @functools.partial(
        pl.kernel, out_type=tuple(out_type), mesh=mesh,
        scratch_types=scratch,
        compiler_params=pltpu.CompilerParams(use_tc_tiling_on_sc=False))
    def k(*refs):
        if with_cnt:
            (table_r, src_r, dst_r, zero_r, zero16_r, ones_r,
             out_r, cnt_r, acc, srcb, dstb, rows, stage, sem,
             acc_c, onesb, stage16) = refs
        else:
            (table_r, src_r, dst_r, zero_r,
             out_r, acc, srcb, dstb, rows, stage, sem) = refs
        cid = lax.axis_index("c")
        sid = lax.axis_index("s")
        r0 = sid * rpt

        def rows_of(fn):
            for j in range(rpt // CH):
                fn(r0 + j * CH, CH)
            if rem:
                @pl.when(sid == _NS - 1)
                def _():
                    fn(_NS * rpt, rem)

        pltpu.sync_copy(zero_r, stage)
        if with_cnt:
            pltpu.sync_copy(zero16_r, stage16)
            pltpu.sync_copy(ones_r, onesb)

        def zero_slice(o, n):
            pltpu.sync_copy(stage.at[pl.ds(0, n)], acc.at[pl.ds(o, n)])
            if with_cnt:
                pltpu.sync_copy(stage16.at[pl.ds(0, n)],
                                acc_c.at[pl.ds(o, n)])
        rows_of(zero_slice)
        plsc.subcore_barrier()

        wbase = (cid * _NS + sid) * epw

        def body(b, carry):
            base = wbase + b * _K
            pltpu.sync_copy(src_r.at[pl.ds(base, _K)], srcb)
            pltpu.sync_copy(dst_r.at[pl.ds(base, _K)], dstb)
            pltpu.async_copy(table_r.at[srcb], rows, sem).wait()
            pltpu.sync_copy(rows, acc.at[dstb], add=True)
            if with_cnt:
                pltpu.sync_copy(onesb, acc_c.at[dstb], add=True)
            return carry

        lax.fori_loop(0, nblk, body, 0)
        plsc.subcore_barrier()

        def write_slice(o, n):
            pltpu.sync_copy(acc.at[pl.ds(o, n)], stage.at[pl.ds(0, n)])
            pltpu.sync_copy(stage.at[pl.ds(0, n)], out_r.at[cid, pl.ds(o, n)])
            if with_cnt:
                pltpu.sync_copy(acc_c.at[pl.ds(o, n)], stage16.at[pl.ds(0, n)])
                pltpu.sync_copy(stage16.at[pl.ds(0, n)],
                                cnt_r.at[cid, pl.ds(o, n)])
        rows_of(write_slice)

    zero = jnp.zeros((CH, D), jnp.float32)
    if with_cnt:
        return k(table, src, dst, zero, jnp.zeros((CH, 16), jnp.float32),
                 jnp.ones((_K, 16), jnp.float32))
    return k(table, src, dst, zero)


def _ln_relu(h, g, b):
    mu = jnp.mean(h, axis=-1, keepdims=True)
    var = jnp.mean((h - mu) ** 2, axis=-1, keepdims=True)
    h = (h - mu) * lax.rsqrt(var + 1e-5) * g + b
    return jnp.maximum(h, 0.0)


def _tc_call(body, n_out, N, R, in_specs_dims, out_dims, *args):
    grid = (N // R,)

    def spec(dims):
        blk = tuple(R if d is None else d for d in dims)
        rowpos = [j for j, d in enumerate(dims) if d is None]
        if rowpos:
            p = rowpos[0]
            return pl.BlockSpec(blk, lambda i, p=p: tuple(
                i if j == p else 0 for j in range(len(blk))))
        return pl.BlockSpec(blk, lambda i: (0,) * len(blk))

    return pl.pallas_call(
        body,
        grid=grid,
        in_specs=[spec(d) for d in in_specs_dims],
        out_specs=[spec(d) for d in out_dims],
        out_shape=[jax.ShapeDtypeStruct(
            tuple(N if d is None else d for d in dims), jnp.float32)
            for dims in out_dims],
    )(*args)


def _dot(a, b):
    return jnp.dot(a, b, preferred_element_type=jnp.float32)


def kernel(z, edge_index, W1l, W1r, b1, W2l, W2r, b2, W3l, W3r, b3,
           W4l, W4r, b4, g1, be1, g2, be2, g3, be3):
    N, D = z.shape
    src = edge_index[0]
    dst = edge_index[1]
    R = 400

    b1r, g1r, be1r = b1[None, :], g1[None, :], be1[None, :]
    b2r, g2r, be2r = b2[None, :], g2[None, :], be2[None, :]
    b3r, g3r, be3r = b3[None, :], g3[None, :], be3[None, :]
    W4lp = jnp.pad(W4l, ((0, 0), (0, 16 - W4l.shape[1])))
    W4rp = jnp.pad(W4r, ((0, 0), (0, 16 - W4r.shape[1])))
    b4p = jnp.pad(b4, (0, 16 - b4.shape[0]))[None, :]

    s1, cntp = _segsum_sc(z, src, dst, with_cnt=True)

    def tc1(s_r, c_r, z_r, wl_r, wr_r, b_r, g_r, be_r, oa_r, ob_r, inv_r):
        c = jnp.maximum(c_r[0] + c_r[1], 1.0)
        inv = 1.0 / c[:, :1]
        mean = (s_r[0] + s_r[1]) * inv
        h = _dot(mean, wl_r[...]) + _dot(z_r[...], wr_r[...]) + b_r[...]
        h = _ln_relu(h, g_r[...], be_r[...])
        oa_r[...] = h[:, :128]
        ob_r[...] = h[:, 128:]
        inv_r[...] = jnp.broadcast_to(inv, inv_r.shape)

    x1a, x1b, inv = _tc_call(
        tc1, 3, N, R,
        [(2, None, 128), (2, None, 16), (None, 128), (128, 256), (128, 256),
         (1, 256), (1, 256), (1, 256)],
        [(None, 128), (None, 128), (None, 16)],
        s1, cntp, z, W1l, W1r, b1r, g1r, be1r)

    sa = _segsum_sc(x1a, src, dst, with_cnt=False)[0]
    sb = _segsum_sc(x1b, src, dst, with_cnt=False)[0]

    def tc2(sa_r, sb_r, inv_r, xa_r, xb_r, wl_r, wr_r, b_r, g_r, be_r,
            w3l_r, o_r, y3_r):
        iv = inv_r[:, :1]
        ma = (sa_r[0] + sa_r[1]) * iv
        mb = (sb_r[0] + sb_r[1]) * iv
        wl = wl_r[...]
        wr = wr_r[...]
        h = (_dot(ma, wl[:128]) + _dot(mb, wl[128:])
             + _dot(xa_r[...], wr[:128]) + _dot(xb_r[...], wr[128:])
             + b_r[...])
        h = _ln_relu(h, g_r[...], be_r[...])
        o_r[...] = h
        y3_r[...] = _dot(h, w3l_r[...])

    x2, y3 = _tc_call(
        tc2, 2, N, R,
        [(2, None, 128), (2, None, 128), (None, 16), (None, 128),
         (None, 128), (256, 256), (256, 256), (1, 256), (1, 256), (1, 256),
         (256, 128)],
        [(None, 256), (None, 128)],
        sa, sb, inv, x1a, x1b, W2l, W2r, b2r, g2r, be2r, W3l)

    s3 = _segsum_sc(y3, src, dst, with_cnt=False)[0]

    def tc3(s_r, inv_r, x2_r, wr_r, b_r, g_r, be_r, o_r):
        mean = (s_r[0] + s_r[1]) * inv_r[:, :1]
        h = mean + _dot(x2_r[...], wr_r[...]) + b_r[...]
        o_r[...] = _ln_relu(h, g_r[...], be_r[...])

    (x3,) = _tc_call(
        tc3, 1, N, R,
        [(2, None, 128), (None, 16), (None, 256), (256, 128), (1, 128),
         (1, 128), (1, 128)],
        [(None, 128)],
        s3, inv, x2, W3r, b3r, g3r, be3r)

    s4 = _segsum_sc(x3, src, dst, with_cnt=False)[0]

    def tc4(s_r, inv_r, x3_r, wl_r, wr_r, b_r, o_r):
        mean = (s_r[0] + s_r[1]) * inv_r[:, :1]
        o_r[...] = (_dot(mean, wl_r[...]) + _dot(x3_r[...], wr_r[...])
                    + b_r[...])

    (o,) = _tc_call(
        tc4, 1, N, R,
        [(2, None, 128), (None, 16), (None, 128), (128, 16), (128, 16),
         (1, 16)],
        [(None, 16)],
        s4, inv, x3, W4lp, W4rp, b4p)

    return o[:, :9]


# speedup vs baseline: 8.2691x; 1.9228x over previous
import functools

import jax
import jax.numpy as jnp
from jax import lax
from jax.experimental import pallas as pl
from jax.experimental.pallas import tpu as pltpu, tpu_sc as plsc

_NC = 2
_NS = 16
_K = 80


def _segsum_sc(table, src, dst, with_cnt):
    N, D = table.shape
    E = src.shape[0]
    NW = _NC * _NS
    epw = E // NW
    nblk = epw // _K
    assert epw * NW == E and nblk * _K == epw
    rpt = (N // _NS) & ~7
    rem = N - _NS * rpt

    CH = 48
    assert rpt % CH == 0 and rem <= CH

    mesh = plsc.VectorSubcoreMesh(core_axis_name="c", subcore_axis_name="s")
    out_type = [jax.ShapeDtypeStruct((_NC, N, D), jnp.float32)]
    scratch = [
        pltpu.VMEM_SHARED((N, D), jnp.float32),
        pltpu.VMEM((2, _K), jnp.int32),
        pltpu.VMEM((2, _K), jnp.int32),
        pltpu.VMEM((2, _K, D), jnp.float32),
        pltpu.VMEM((CH, D), jnp.float32),
        pltpu.SemaphoreType.DMA((2,)),
        pltpu.SemaphoreType.DMA((2,)),
        pltpu.SemaphoreType.DMA((2,)),
    ]
    if with_cnt:
        out_type.append(jax.ShapeDtypeStruct((_NC, N, 16), jnp.float32))
        scratch += [
            pltpu.VMEM_SHARED((N, 16), jnp.float32),
            pltpu.VMEM((_K, 16), jnp.float32),
            pltpu.VMEM((CH, 16), jnp.float32),
            pltpu.SemaphoreType.DMA((2,)),
        ]

    @functools.partial(
        pl.kernel, out_type=tuple(out_type), mesh=mesh,
        scratch_types=scratch,
        compiler_params=pltpu.CompilerParams(use_tc_tiling_on_sc=False))
    def k(*refs):
        if with_cnt:
            (table_r, src_r, dst_r, zero_r, zero16_r, ones_r,
             out_r, cnt_r, acc, srcb, dstb, rows, stage, isem, gsem, ssem,
             acc_c, onesb, stage16, csem) = refs
        else:
            (table_r, src_r, dst_r, zero_r,
             out_r, acc, srcb, dstb, rows, stage, isem, gsem, ssem) = refs
        cid = lax.axis_index("c")
        sid = lax.axis_index("s")
        r0 = sid * rpt

        def rows_of(fn):
            for j in range(rpt // CH):
                fn(r0 + j * CH, CH)
            if rem:
                @pl.when(sid == _NS - 1)
                def _():
                    fn(_NS * rpt, rem)

        pltpu.sync_copy(zero_r, stage)
        if with_cnt:
            pltpu.sync_copy(zero16_r, stage16)
            pltpu.sync_copy(ones_r, onesb)

        def zero_slice(o, n):
            pltpu.sync_copy(stage.at[pl.ds(0, n)], acc.at[pl.ds(o, n)])
            if with_cnt:
                pltpu.sync_copy(stage16.at[pl.ds(0, n)],
                                acc_c.at[pl.ds(o, n)])
        rows_of(zero_slice)
        plsc.subcore_barrier()

        wbase = (cid * _NS + sid) * epw

        def issue_idx(b, s):
            base = wbase + b * _K
            pltpu.async_copy(src_r.at[pl.ds(base, _K)], srcb.at[s],
                             isem.at[s])
            pltpu.async_copy(dst_r.at[pl.ds(base, _K)], dstb.at[s],
                             isem.at[s])

        def wait_idx(s):
            pltpu.make_async_copy(src_r.at[pl.ds(0, _K)], srcb.at[s],
                                  isem.at[s]).wait()
            pltpu.make_async_copy(dst_r.at[pl.ds(0, _K)], dstb.at[s],
                                  isem.at[s]).wait()

        def issue_gather(s):
            pltpu.async_copy(table_r.at[srcb.at[s]], rows.at[s], gsem.at[s])

        def wait_gather(s):
            pltpu.make_async_copy(table_r.at[srcb.at[s]], rows.at[s],
                                  gsem.at[s]).wait()

        def issue_scatter(s):
            pltpu.async_copy(rows.at[s], acc.at[dstb.at[s]], ssem.at[s],
                             add=True)
            if with_cnt:
                pltpu.async_copy(onesb, acc_c.at[dstb.at[s]], csem.at[s],
                                 add=True)

        def wait_scatter(s):
            pltpu.make_async_copy(rows.at[s], acc.at[dstb.at[s]],
                                  ssem.at[s]).wait()
            if with_cnt:
                pltpu.make_async_copy(onesb, acc_c.at[dstb.at[s]],
                                      csem.at[s]).wait()

        issue_idx(0, 0)
        wait_idx(0)
        issue_gather(0)

        def body(b, carry):
            u = lax.rem(b, 2)
            ou = 1 - u

            @pl.when(b > 0)
            def _():
                wait_scatter(ou)
            issue_idx(lax.rem(b + 1, nblk), ou)
            wait_idx(ou)
            issue_gather(ou)
            wait_gather(u)
            issue_scatter(u)
            return carry

        lax.fori_loop(0, nblk, body, 0)
        wait_scatter((nblk - 1) % 2)
        wait_gather(nblk % 2)
        plsc.subcore_barrier()

        def write_slice(o, n):
            pltpu.sync_copy(acc.at[pl.ds(o, n)], stage.at[pl.ds(0, n)])
            pltpu.sync_copy(stage.at[pl.ds(0, n)], out_r.at[cid, pl.ds(o, n)])
            if with_cnt:
                pltpu.sync_copy(acc_c.at[pl.ds(o, n)], stage16.at[pl.ds(0, n)])
                pltpu.sync_copy(stage16.at[pl.ds(0, n)],
                                cnt_r.at[cid, pl.ds(o, n)])
        rows_of(write_slice)

    zero = jnp.zeros((CH, D), jnp.float32)
    if with_cnt:
        return k(table, src, dst, zero, jnp.zeros((CH, 16), jnp.float32),
                 jnp.ones((_K, 16), jnp.float32))
    return k(table, src, dst, zero)


def _ln_relu(h, g, b):
    mu = jnp.mean(h, axis=-1, keepdims=True)
    var = jnp.mean((h - mu) ** 2, axis=-1, keepdims=True)
    h = (h - mu) * lax.rsqrt(var + 1e-5) * g + b
    return jnp.maximum(h, 0.0)


def _tc_call(body, n_out, N, R, in_specs_dims, out_dims, *args):
    grid = (N // R,)

    def spec(dims):
        blk = tuple(R if d is None else d for d in dims)
        rowpos = [j for j, d in enumerate(dims) if d is None]
        if rowpos:
            p = rowpos[0]
            return pl.BlockSpec(blk, lambda i, p=p: tuple(
                i if j == p else 0 for j in range(len(blk))))
        return pl.BlockSpec(blk, lambda i: (0,) * len(blk))

    return pl.pallas_call(
        body,
        grid=grid,
        in_specs=[spec(d) for d in in_specs_dims],
        out_specs=[spec(d) for d in out_dims],
        out_shape=[jax.ShapeDtypeStruct(
            tuple(N if d is None else d for d in dims), jnp.float32)
            for dims in out_dims],
    )(*args)


def _dot(a, b):
    return jnp.dot(a, b, preferred_element_type=jnp.float32)


def kernel(z, edge_index, W1l, W1r, b1, W2l, W2r, b2, W3l, W3r, b3,
           W4l, W4r, b4, g1, be1, g2, be2, g3, be3):
    N, D = z.shape
    src = edge_index[0]
    dst = edge_index[1]
    R = 400

    b1r, g1r, be1r = b1[None, :], g1[None, :], be1[None, :]
    b2r, g2r, be2r = b2[None, :], g2[None, :], be2[None, :]
    b3r, g3r, be3r = b3[None, :], g3[None, :], be3[None, :]
    W4lp = jnp.pad(W4l, ((0, 0), (0, 16 - W4l.shape[1])))
    W4rp = jnp.pad(W4r, ((0, 0), (0, 16 - W4r.shape[1])))
    b4p = jnp.pad(b4, (0, 16 - b4.shape[0]))[None, :]

    s1, cntp = _segsum_sc(z, src, dst, with_cnt=True)

    def tc1(s_r, c_r, z_r, wl_r, wr_r, b_r, g_r, be_r, oa_r, ob_r, inv_r):
        c = jnp.maximum(c_r[0] + c_r[1], 1.0)
        inv = 1.0 / c[:, :1]
        mean = (s_r[0] + s_r[1]) * inv
        h = _dot(mean, wl_r[...]) + _dot(z_r[...], wr_r[...]) + b_r[...]
        h = _ln_relu(h, g_r[...], be_r[...])
        oa_r[...] = h[:, :128]
        ob_r[...] = h[:, 128:]
        inv_r[...] = jnp.broadcast_to(inv, inv_r.shape)

    x1a, x1b, inv = _tc_call(
        tc1, 3, N, R,
        [(2, None, 128), (2, None, 16), (None, 128), (128, 256), (128, 256),
         (1, 256), (1, 256), (1, 256)],
        [(None, 128), (None, 128), (None, 16)],
        s1, cntp, z, W1l, W1r, b1r, g1r, be1r)

    sa = _segsum_sc(x1a, src, dst, with_cnt=False)[0]
    sb = _segsum_sc(x1b, src, dst, with_cnt=False)[0]

    def tc2(sa_r, sb_r, inv_r, xa_r, xb_r, wl_r, wr_r, b_r, g_r, be_r,
            w3l_r, o_r, y3_r):
        iv = inv_r[:, :1]
        ma = (sa_r[0] + sa_r[1]) * iv
        mb = (sb_r[0] + sb_r[1]) * iv
        wl = wl_r[...]
        wr = wr_r[...]
        h = (_dot(ma, wl[:128]) + _dot(mb, wl[128:])
             + _dot(xa_r[...], wr[:128]) + _dot(xb_r[...], wr[128:])
             + b_r[...])
        h = _ln_relu(h, g_r[...], be_r[...])
        o_r[...] = h
        y3_r[...] = _dot(h, w3l_r[...])

    x2, y3 = _tc_call(
        tc2, 2, N, R,
        [(2, None, 128), (2, None, 128), (None, 16), (None, 128),
         (None, 128), (256, 256), (256, 256), (1, 256), (1, 256), (1, 256),
         (256, 128)],
        [(None, 256), (None, 128)],
        sa, sb, inv, x1a, x1b, W2l, W2r, b2r, g2r, be2r, W3l)

    s3 = _segsum_sc(y3, src, dst, with_cnt=False)[0]

    def tc3(s_r, inv_r, x2_r, wr_r, b_r, g_r, be_r, o_r):
        mean = (s_r[0] + s_r[1]) * inv_r[:, :1]
        h = mean + _dot(x2_r[...], wr_r[...]) + b_r[...]
        o_r[...] = _ln_relu(h, g_r[...], be_r[...])

    (x3,) = _tc_call(
        tc3, 1, N, R,
        [(2, None, 128), (None, 16), (None, 256), (256, 128), (1, 128),
         (1, 128), (1, 128)],
        [(None, 128)],
        s3, inv, x2, W3r, b3r, g3r, be3r)

    s4 = _segsum_sc(x3, src, dst, with_cnt=False)[0]

    def tc4(s_r, inv_r, x3_r, wl_r, wr_r, b_r, o_r):
        mean = (s_r[0] + s_r[1]) * inv_r[:, :1]
        o_r[...] = (_dot(mean, wl_r[...]) + _dot(x3_r[...], wr_r[...])
                    + b_r[...])

    (o,) = _tc_call(
        tc4, 1, N, R,
        [(2, None, 128), (None, 16), (None, 128), (128, 16), (128, 16),
         (1, 16)],
        [(None, 16)],
        s4, inv, x3, W4lp, W4rp, b4p)

    return o[:, :9]


# speedup vs baseline: 10.8690x; 1.3144x over previous
import functools

import jax
import jax.numpy as jnp
from jax import lax
from jax.experimental import pallas as pl
from jax.experimental.pallas import tpu as pltpu, tpu_sc as plsc

_NC = 2
_NS = 16
_K = 80


def _segsum_sc(table, src, dst, with_cnt):
    N, D = table.shape
    E = src.shape[0]
    NW = _NC * _NS
    epw = E // NW
    nblk = epw // _K
    assert epw * NW == E and nblk * _K == epw
    rpt = (N // _NS) & ~7
    rem = N - _NS * rpt

    NB = 3 if with_cnt else 4

    mesh = plsc.VectorSubcoreMesh(core_axis_name="c", subcore_axis_name="s")
    out_type = [jax.ShapeDtypeStruct((_NC, N, D), jnp.float32)]
    scratch = [
        pltpu.VMEM_SHARED((N, D), jnp.float32),
        pltpu.VMEM((NB, _K), jnp.int32),
        pltpu.VMEM((NB, _K), jnp.int32),
        pltpu.VMEM((NB, _K, D), jnp.float32),
        pltpu.SemaphoreType.DMA((NB,)),
        pltpu.SemaphoreType.DMA((NB,)),
        pltpu.SemaphoreType.DMA((NB,)),
    ]
    if with_cnt:
        out_type.append(jax.ShapeDtypeStruct((_NC, N, 16), jnp.float32))
        scratch += [
            pltpu.VMEM_SHARED((N, 16), jnp.float32),
            pltpu.VMEM((_K, 16), jnp.float32),
            pltpu.SemaphoreType.DMA((NB,)),
        ]

    @functools.partial(
        pl.kernel, out_type=tuple(out_type), mesh=mesh,
        scratch_types=scratch,
        compiler_params=pltpu.CompilerParams(use_tc_tiling_on_sc=False))
    def k(*refs):
        if with_cnt:
            (table_r, src_r, dst_r, zero_r, zero16_r, ones_r,
             out_r, cnt_r, acc, srcb, dstb, rows, isem, gsem, ssem,
             acc_c, onesb, csem) = refs
        else:
            (table_r, src_r, dst_r, zero_r,
             out_r, acc, srcb, dstb, rows, isem, gsem, ssem) = refs
        cid = lax.axis_index("c")
        sid = lax.axis_index("s")
        r0 = sid * rpt

        def rows_of(fn):
            fn(r0, rpt)
            if rem:
                @pl.when(sid == _NS - 1)
                def _():
                    fn(_NS * rpt, rem)

        def zero_slice(o, n):
            pltpu.sync_copy(zero_r.at[pl.ds(0, n)], acc.at[pl.ds(o, n)])
            if with_cnt:
                pltpu.sync_copy(zero16_r.at[pl.ds(0, n)],
                                acc_c.at[pl.ds(o, n)])
        rows_of(zero_slice)
        if with_cnt:
            pltpu.sync_copy(ones_r, onesb)
        plsc.subcore_barrier()

        wbase = (cid * _NS + sid) * epw

        def issue_idx(b, s):
            base = wbase + b * _K
            pltpu.async_copy(src_r.at[pl.ds(base, _K)], srcb.at[s],
                             isem.at[s])
            pltpu.async_copy(dst_r.at[pl.ds(base, _K)], dstb.at[s],
                             isem.at[s])

        def wait_idx(s):
            pltpu.make_async_copy(src_r.at[pl.ds(0, _K)], srcb.at[s],
                                  isem.at[s]).wait()
            pltpu.make_async_copy(dst_r.at[pl.ds(0, _K)], dstb.at[s],
                                  isem.at[s]).wait()

        def issue_gather(s):
            pltpu.async_copy(table_r.at[srcb.at[s]], rows.at[s], gsem.at[s])

        def wait_gather(s):
            pltpu.make_async_copy(table_r.at[srcb.at[s]], rows.at[s],
                                  gsem.at[s]).wait()

        def issue_scatter(s):
            pltpu.async_copy(rows.at[s], acc.at[dstb.at[s]], ssem.at[s],
                             add=True)
            if with_cnt:
                pltpu.async_copy(onesb, acc_c.at[dstb.at[s]], csem.at[s],
                                 add=True)

        def wait_scatter(s):
            pltpu.make_async_copy(rows.at[s], acc.at[dstb.at[s]],
                                  ssem.at[s]).wait()
            if with_cnt:
                pltpu.make_async_copy(onesb, acc_c.at[dstb.at[s]],
                                      csem.at[s]).wait()

        issue_idx(0, 0)
        issue_idx(1 % nblk, 1 % NB)
        wait_idx(0)
        issue_gather(0)

        def body(b, carry):
            u = lax.rem(b, NB)
            u1 = lax.rem(b + 1, NB)
            u2 = lax.rem(b + 2, NB)

            @pl.when(b >= NB - 2)
            def _():
                wait_scatter(u2)
            issue_idx(lax.rem(b + 2, nblk), u2)
            wait_idx(u1)
            issue_gather(u1)
            wait_gather(u)
            issue_scatter(u)
            return carry

        lax.fori_loop(0, nblk, body, 0)
        for j in range(nblk + 2 - NB, nblk):
            wait_scatter(j % NB)
        wait_gather(nblk % NB)
        wait_idx((nblk + 1) % NB)
        plsc.subcore_barrier()

        def write_slice(o, n):
            pltpu.sync_copy(acc.at[pl.ds(o, n)], out_r.at[cid, pl.ds(o, n)])
            if with_cnt:
                pltpu.sync_copy(acc_c.at[pl.ds(o, n)],
                                cnt_r.at[cid, pl.ds(o, n)])
        rows_of(write_slice)

    zero = jnp.zeros((rpt, D), jnp.float32)
    if with_cnt:
        return k(table, src, dst, zero, jnp.zeros((rpt, 16), jnp.float32),
                 jnp.ones((_K, 16), jnp.float32))
    return k(table, src, dst, zero)


def _ln_relu(h, g, b):
    mu = jnp.mean(h, axis=-1, keepdims=True)
    var = jnp.mean((h - mu) ** 2, axis=-1, keepdims=True)
    h = (h - mu) * lax.rsqrt(var + 1e-5) * g + b
    return jnp.maximum(h, 0.0)


def _tc_call(body, n_out, N, R, in_specs_dims, out_dims, *args):
    grid = (N // R,)

    def spec(dims):
        blk = tuple(R if d is None else d for d in dims)
        rowpos = [j for j, d in enumerate(dims) if d is None]
        if rowpos:
            p = rowpos[0]
            return pl.BlockSpec(blk, lambda i, p=p: tuple(
                i if j == p else 0 for j in range(len(blk))))
        return pl.BlockSpec(blk, lambda i: (0,) * len(blk))

    return pl.pallas_call(
        body,
        grid=grid,
        in_specs=[spec(d) for d in in_specs_dims],
        out_specs=[spec(d) for d in out_dims],
        out_shape=[jax.ShapeDtypeStruct(
            tuple(N if d is None else d for d in dims), jnp.float32)
            for dims in out_dims],
    )(*args)


def _dot(a, b):
    return jnp.dot(a, b, preferred_element_type=jnp.float32)


def kernel(z, edge_index, W1l, W1r, b1, W2l, W2r, b2, W3l, W3r, b3,
           W4l, W4r, b4, g1, be1, g2, be2, g3, be3):
    N, D = z.shape
    src = edge_index[0]
    dst = edge_index[1]
    R = 400

    b1r, g1r, be1r = b1[None, :], g1[None, :], be1[None, :]
    b2r, g2r, be2r = b2[None, :], g2[None, :], be2[None, :]
    b3r, g3r, be3r = b3[None, :], g3[None, :], be3[None, :]
    W4lp = jnp.pad(W4l, ((0, 0), (0, 16 - W4l.shape[1])))
    W4rp = jnp.pad(W4r, ((0, 0), (0, 16 - W4r.shape[1])))
    b4p = jnp.pad(b4, (0, 16 - b4.shape[0]))[None, :]

    s1, cntp = _segsum_sc(z, src, dst, with_cnt=True)

    def tc1(s_r, c_r, z_r, wl_r, wr_r, b_r, g_r, be_r, oa_r, ob_r, inv_r):
        c = jnp.maximum(c_r[0] + c_r[1], 1.0)
        inv = 1.0 / c[:, :1]
        mean = (s_r[0] + s_r[1]) * inv
        h = _dot(mean, wl_r[...]) + _dot(z_r[...], wr_r[...]) + b_r[...]
        h = _ln_relu(h, g_r[...], be_r[...])
        oa_r[...] = h[:, :128]
        ob_r[...] = h[:, 128:]
        inv_r[...] = jnp.broadcast_to(inv, inv_r.shape)

    x1a, x1b, inv = _tc_call(
        tc1, 3, N, R,
        [(2, None, 128), (2, None, 16), (None, 128), (128, 256), (128, 256),
         (1, 256), (1, 256), (1, 256)],
        [(None, 128), (None, 128), (None, 16)],
        s1, cntp, z, W1l, W1r, b1r, g1r, be1r)

    sa = _segsum_sc(x1a, src, dst, with_cnt=False)[0]
    sb = _segsum_sc(x1b, src, dst, with_cnt=False)[0]

    def tc2(sa_r, sb_r, inv_r, xa_r, xb_r, wl_r, wr_r, b_r, g_r, be_r,
            w3l_r, o_r, y3_r):
        iv = inv_r[:, :1]
        ma = (sa_r[0] + sa_r[1]) * iv
        mb = (sb_r[0] + sb_r[1]) * iv
        wl = wl_r[...]
        wr = wr_r[...]
        h = (_dot(ma, wl[:128]) + _dot(mb, wl[128:])
             + _dot(xa_r[...], wr[:128]) + _dot(xb_r[...], wr[128:])
             + b_r[...])
        h = _ln_relu(h, g_r[...], be_r[...])
        o_r[...] = h
        y3_r[...] = _dot(h, w3l_r[...])

    x2, y3 = _tc_call(
        tc2, 2, N, R,
        [(2, None, 128), (2, None, 128), (None, 16), (None, 128),
         (None, 128), (256, 256), (256, 256), (1, 256), (1, 256), (1, 256),
         (256, 128)],
        [(None, 256), (None, 128)],
        sa, sb, inv, x1a, x1b, W2l, W2r, b2r, g2r, be2r, W3l)

    s3 = _segsum_sc(y3, src, dst, with_cnt=False)[0]

    def tc3(s_r, inv_r, x2_r, wr_r, b_r, g_r, be_r, o_r):
        mean = (s_r[0] + s_r[1]) * inv_r[:, :1]
        h = mean + _dot(x2_r[...], wr_r[...]) + b_r[...]
        o_r[...] = _ln_relu(h, g_r[...], be_r[...])

    (x3,) = _tc_call(
        tc3, 1, N, R,
        [(2, None, 128), (None, 16), (None, 256), (256, 128), (1, 128),
         (1, 128), (1, 128)],
        [(None, 128)],
        s3, inv, x2, W3r, b3r, g3r, be3r)

    s4 = _segsum_sc(x3, src, dst, with_cnt=False)[0]

    def tc4(s_r, inv_r, x3_r, wl_r, wr_r, b_r, o_r):
        mean = (s_r[0] + s_r[1]) * inv_r[:, :1]
        o_r[...] = (_dot(mean, wl_r[...]) + _dot(x3_r[...], wr_r[...])
                    + b_r[...])

    (o,) = _tc_call(
        tc4, 1, N, R,
        [(2, None, 128), (None, 16), (None, 128), (128, 16), (128, 16),
         (1, 16)],
        [(None, 16)],
        s4, inv, x3, W4lp, W4rp, b4p)

    return o[:, :9]


# speedup vs baseline: 11.8467x; 1.0899x over previous
import functools

import jax
import jax.numpy as jnp
from jax import lax
from jax.experimental import pallas as pl
from jax.experimental.pallas import tpu as pltpu, tpu_sc as plsc

_NC = 2
_NS = 16
_K = 80


def _segsum_sc(table, edge_index, with_cnt):
    N, D = table.shape
    E = edge_index.shape[1]
    NW = _NC * _NS
    epw = E // NW
    nblk = epw // _K
    assert epw * NW == E and nblk * _K == epw
    rpt = (N // _NS) & ~7
    rem = N - _NS * rpt

    NB = 3 if with_cnt else 4

    mesh = plsc.VectorSubcoreMesh(core_axis_name="c", subcore_axis_name="s")
    out_type = [jax.ShapeDtypeStruct((_NC, N, D), jnp.float32)]
    scratch = [
        pltpu.VMEM_SHARED((N, D), jnp.float32),
        pltpu.VMEM((NB, _K), jnp.int32),
        pltpu.VMEM((NB, _K), jnp.int32),
        pltpu.VMEM((NB, _K, D), jnp.float32),
        pltpu.SemaphoreType.DMA((NB,)),
        pltpu.SemaphoreType.DMA((NB,)),
        pltpu.SemaphoreType.DMA((NB,)),
    ]
    if with_cnt:
        out_type.append(jax.ShapeDtypeStruct((_NC, _NS, N), jnp.float32))
        scratch += [
            pltpu.VMEM((N,), jnp.float32),
        ]

    @functools.partial(
        pl.kernel, out_type=tuple(out_type), mesh=mesh,
        scratch_types=scratch,
        compiler_params=pltpu.CompilerParams(use_tc_tiling_on_sc=False,
                                             needs_layout_passes=False))
    def k(*refs):
        if with_cnt:
            (table_r, ei_r, zero_r, zcnt_r,
             out_r, cnt_r, acc, srcb, dstb, rows, isem, gsem, ssem,
             cntloc) = refs
        else:
            (table_r, ei_r, zero_r,
             out_r, acc, srcb, dstb, rows, isem, gsem, ssem) = refs
        cid = lax.axis_index("c")
        sid = lax.axis_index("s")
        r0 = sid * rpt

        def rows_of(fn):
            fn(r0, rpt)
            if rem:
                @pl.when(sid == _NS - 1)
                def _():
                    fn(_NS * rpt, rem)

        def zero_slice(o, n):
            pltpu.sync_copy(zero_r.at[pl.ds(0, n)], acc.at[pl.ds(o, n)])
        rows_of(zero_slice)
        if with_cnt:
            pltpu.sync_copy(zcnt_r, cntloc)
        plsc.subcore_barrier()

        wbase = (cid * _NS + sid) * epw

        def issue_idx(b, s):
            base = wbase + b * _K
            pltpu.async_copy(ei_r.at[0, pl.ds(base, _K)], srcb.at[s],
                             isem.at[s])
            pltpu.async_copy(ei_r.at[1, pl.ds(base, _K)], dstb.at[s],
                             isem.at[s])

        def wait_idx(s):
            pltpu.make_async_copy(ei_r.at[0, pl.ds(0, _K)], srcb.at[s],
                                  isem.at[s]).wait()
            pltpu.make_async_copy(ei_r.at[1, pl.ds(0, _K)], dstb.at[s],
                                  isem.at[s]).wait()

        def issue_gather(s):
            pltpu.async_copy(table_r.at[srcb.at[s]], rows.at[s], gsem.at[s])

        def wait_gather(s):
            pltpu.make_async_copy(table_r.at[srcb.at[s]], rows.at[s],
                                  gsem.at[s]).wait()

        def issue_scatter(s):
            pltpu.async_copy(rows.at[s], acc.at[dstb.at[s]], ssem.at[s],
                             add=True)

        def wait_scatter(s):
            pltpu.make_async_copy(rows.at[s], acc.at[dstb.at[s]],
                                  ssem.at[s]).wait()

        ones16 = jnp.ones((16,), jnp.float32)

        def count_block(s):
            if with_cnt:
                for j in range(_K // 16):
                    iv = dstb.at[s][pl.ds(j * 16, 16)]
                    plsc.addupdate_scatter(cntloc, [iv], ones16)

        issue_idx(0, 0)
        issue_idx(1 % nblk, 1 % NB)
        wait_idx(0)
        issue_gather(0)

        def body(b, carry):
            u = lax.rem(b, NB)
            u1 = lax.rem(b + 1, NB)
            u2 = lax.rem(b + 2, NB)

            @pl.when(b >= NB - 2)
            def _():
                wait_scatter(u2)
            issue_idx(lax.rem(b + 2, nblk), u2)
            wait_idx(u1)
            issue_gather(u1)
            wait_gather(u)
            issue_scatter(u)
            count_block(u)
            return carry

        lax.fori_loop(0, nblk, body, 0)
        for j in range(nblk + 2 - NB, nblk):
            wait_scatter(j % NB)
        wait_gather(nblk % NB)
        wait_idx((nblk + 1) % NB)
        plsc.subcore_barrier()

        def write_slice(o, n):
            pltpu.sync_copy(acc.at[pl.ds(o, n)], out_r.at[cid, pl.ds(o, n)])
        rows_of(write_slice)
        if with_cnt:
            pltpu.sync_copy(cntloc, cnt_r.at[cid, sid])

    zero = jnp.zeros((rpt, D), jnp.float32)
    if with_cnt:
        return k(table, edge_index, zero, jnp.zeros((N,), jnp.float32))
    return k(table, edge_index, zero)


def _ln_relu(h, g, b):
    mu = jnp.mean(h, axis=-1, keepdims=True)
    var = jnp.mean((h - mu) ** 2, axis=-1, keepdims=True)
    h = (h - mu) * lax.rsqrt(var + 1e-5) * g + b
    return jnp.maximum(h, 0.0)


def _tc_call(body, n_out, N, R, in_specs_dims, out_dims, *args):
    grid = (N // R,)

    def spec(dims):
        blk = tuple(R if d is None else d for d in dims)
        rowpos = [j for j, d in enumerate(dims) if d is None]
        if rowpos:
            p = rowpos[0]
            return pl.BlockSpec(blk, lambda i, p=p: tuple(
                i if j == p else 0 for j in range(len(blk))))
        return pl.BlockSpec(blk, lambda i: (0,) * len(blk))

    return pl.pallas_call(
        body,
        grid=grid,
        in_specs=[spec(d) for d in in_specs_dims],
        out_specs=[spec(d) for d in out_dims],
        out_shape=[jax.ShapeDtypeStruct(
            tuple(N if d is None else d for d in dims), jnp.float32)
            for dims in out_dims],
    )(*args)


def _dot(a, b):
    return jnp.dot(a, b, preferred_element_type=jnp.float32)


def kernel(z, edge_index, W1l, W1r, b1, W2l, W2r, b2, W3l, W3r, b3,
           W4l, W4r, b4, g1, be1, g2, be2, g3, be3):
    N, D = z.shape
    R = 1000

    b1r, g1r, be1r = b1[None, :], g1[None, :], be1[None, :]
    b2r, g2r, be2r = b2[None, :], g2[None, :], be2[None, :]
    b3r, g3r, be3r = b3[None, :], g3[None, :], be3[None, :]
    W4lp = jnp.pad(W4l, ((0, 0), (0, 16 - W4l.shape[1])))
    W4rp = jnp.pad(W4r, ((0, 0), (0, 16 - W4r.shape[1])))
    b4p = jnp.pad(b4, (0, 16 - b4.shape[0]))[None, :]

    s1, cntp = _segsum_sc(z, edge_index, with_cnt=True)
    cntT = cntp.reshape(_NC * _NS, N).T

    def tc1(s_r, c_r, z_r, wl_r, wr_r, b_r, g_r, be_r, oa_r, ob_r, inv_r):
        c = jnp.maximum(jnp.sum(c_r[...], axis=1), 1.0)
        inv = (1.0 / c)[:, None]
        mean = (s_r[0] + s_r[1]) * inv
        h = _dot(mean, wl_r[...]) + _dot(z_r[...], wr_r[...]) + b_r[...]
        h = _ln_relu(h, g_r[...], be_r[...])
        oa_r[...] = h[:, :128]
        ob_r[...] = h[:, 128:]
        inv_r[...] = jnp.broadcast_to(inv, inv_r.shape)

    x1a, x1b, inv = _tc_call(
        tc1, 3, N, R,
        [(2, None, 128), (None, 32), (None, 128), (128, 256), (128, 256),
         (1, 256), (1, 256), (1, 256)],
        [(None, 128), (None, 128), (None, 16)],
        s1, cntT, z, W1l, W1r, b1r, g1r, be1r)

    sa = _segsum_sc(x1a, edge_index, with_cnt=False)[0]
    sb = _segsum_sc(x1b, edge_index, with_cnt=False)[0]

    def tc2(sa_r, sb_r, inv_r, xa_r, xb_r, wl_r, wr_r, b_r, g_r, be_r,
            w3l_r, o_r, y3_r):
        iv = inv_r[:, :1]
        ma = (sa_r[0] + sa_r[1]) * iv
        mb = (sb_r[0] + sb_r[1]) * iv
        wl = wl_r[...]
        wr = wr_r[...]
        h = (_dot(ma, wl[:128]) + _dot(mb, wl[128:])
             + _dot(xa_r[...], wr[:128]) + _dot(xb_r[...], wr[128:])
             + b_r[...])
        h = _ln_relu(h, g_r[...], be_r[...])
        o_r[...] = h
        y3_r[...] = _dot(h, w3l_r[...])

    x2, y3 = _tc_call(
        tc2, 2, N, R,
        [(2, None, 128), (2, None, 128), (None, 16), (None, 128),
         (None, 128), (256, 256), (256, 256), (1, 256), (1, 256), (1, 256),
         (256, 128)],
        [(None, 256), (None, 128)],
        sa, sb, inv, x1a, x1b, W2l, W2r, b2r, g2r, be2r, W3l)

    s3 = _segsum_sc(y3, edge_index, with_cnt=False)[0]

    def tc3(s_r, inv_r, x2_r, wr_r, b_r, g_r, be_r, o_r):
        mean = (s_r[0] + s_r[1]) * inv_r[:, :1]
        h = mean + _dot(x2_r[...], wr_r[...]) + b_r[...]
        o_r[...] = _ln_relu(h, g_r[...], be_r[...])

    (x3,) = _tc_call(
        tc3, 1, N, R,
        [(2, None, 128), (None, 16), (None, 256), (256, 128), (1, 128),
         (1, 128), (1, 128)],
        [(None, 128)],
        s3, inv, x2, W3r, b3r, g3r, be3r)

    s4 = _segsum_sc(x3, edge_index, with_cnt=False)[0]

    def tc4(s_r, inv_r, x3_r, wl_r, wr_r, b_r, o_r):
        mean = (s_r[0] + s_r[1]) * inv_r[:, :1]
        o_r[...] = (_dot(mean, wl_r[...]) + _dot(x3_r[...], wr_r[...])
                    + b_r[...])

    (o,) = _tc_call(
        tc4, 1, N, R,
        [(2, None, 128), (None, 16), (None, 128), (128, 16), (128, 16),
         (1, 16)],
        [(None, 16)],
        s4, inv, x3, W4lp, W4rp, b4p)

    return o[:, :9]


# speedup vs baseline: 12.3627x; 1.0436x over previous
import functools

import jax
import jax.numpy as jnp
from jax import lax
from jax.experimental import pallas as pl
from jax.experimental.pallas import tpu as pltpu, tpu_sc as plsc

_NC = 2
_NS = 16
_K = 80


def _segsum_sc(table, edge_index, with_cnt):
    N, D = table.shape
    E = edge_index.shape[1]
    NW = _NC * _NS
    epw = E // NW
    nblk = epw // _K
    assert epw * NW == E and nblk * _K == epw
    rpt = (N // _NS) & ~7
    rem = N - _NS * rpt

    NB = 3 if with_cnt else 4

    mesh = plsc.VectorSubcoreMesh(core_axis_name="c", subcore_axis_name="s")
    out_type = [jax.ShapeDtypeStruct((_NC, N, D), jnp.float32)]
    scratch = [
        pltpu.VMEM_SHARED((N, D), jnp.float32),
        pltpu.VMEM((NB, _K), jnp.int32),
        pltpu.VMEM((NB, _K), jnp.int32),
        pltpu.VMEM((NB, _K, D), jnp.float32),
        pltpu.SemaphoreType.DMA((NB,)),
        pltpu.SemaphoreType.DMA((NB,)),
        pltpu.SemaphoreType.DMA((NB,)),
    ]
    if with_cnt:
        out_type.append(jax.ShapeDtypeStruct((_NC, _NS, N), jnp.float32))
        scratch += [
            pltpu.VMEM((N,), jnp.float32),
        ]

    @functools.partial(
        pl.kernel, out_type=tuple(out_type), mesh=mesh,
        scratch_types=scratch,
        compiler_params=pltpu.CompilerParams(use_tc_tiling_on_sc=False,
                                             needs_layout_passes=False))
    def k(*refs):
        if with_cnt:
            (table_r, ei_r, zero_r, zcnt_r,
             out_r, cnt_r, acc, srcb, dstb, rows, isem, gsem, ssem,
             cntloc) = refs
        else:
            (table_r, ei_r, zero_r,
             out_r, acc, srcb, dstb, rows, isem, gsem, ssem) = refs
        cid = lax.axis_index("c")
        sid = lax.axis_index("s")
        r0 = sid * rpt

        def rows_of(fn):
            fn(r0, rpt)
            if rem:
                @pl.when(sid == _NS - 1)
                def _():
                    fn(_NS * rpt, rem)

        def zero_slice(o, n):
            pltpu.sync_copy(zero_r.at[pl.ds(0, n)], acc.at[pl.ds(o, n)])
        rows_of(zero_slice)
        if with_cnt:
            pltpu.sync_copy(zcnt_r, cntloc)
        plsc.subcore_barrier()

        wbase = (cid * _NS + sid) * epw

        def issue_idx(b, s):
            base = wbase + b * _K
            pltpu.async_copy(ei_r.at[0, pl.ds(base, _K)], srcb.at[s],
                             isem.at[s])
            pltpu.async_copy(ei_r.at[1, pl.ds(base, _K)], dstb.at[s],
                             isem.at[s])

        def wait_idx(s):
            pltpu.make_async_copy(ei_r.at[0, pl.ds(0, _K)], srcb.at[s],
                                  isem.at[s]).wait()
            pltpu.make_async_copy(ei_r.at[1, pl.ds(0, _K)], dstb.at[s],
                                  isem.at[s]).wait()

        def issue_gather(s):
            pltpu.async_copy(table_r.at[srcb.at[s]], rows.at[s], gsem.at[s])

        def wait_gather(s):
            pltpu.make_async_copy(table_r.at[srcb.at[s]], rows.at[s],
                                  gsem.at[s]).wait()

        def issue_scatter(s):
            pltpu.async_copy(rows.at[s], acc.at[dstb.at[s]], ssem.at[s],
                             add=True)

        def wait_scatter(s):
            pltpu.make_async_copy(rows.at[s], acc.at[dstb.at[s]],
                                  ssem.at[s]).wait()

        ones16 = jnp.ones((16,), jnp.float32)

        def count_block(s):
            if with_cnt:
                for j in range(_K // 16):
                    iv = dstb.at[s][pl.ds(j * 16, 16)]
                    plsc.addupdate_scatter(cntloc, [iv], ones16)

        issue_idx(0, 0)
        issue_idx(1 % nblk, 1 % NB)
        wait_idx(0)
        issue_gather(0)

        def body(b, carry):
            u = lax.rem(b, NB)
            u1 = lax.rem(b + 1, NB)
            u2 = lax.rem(b + 2, NB)

            @pl.when(b >= NB - 2)
            def _():
                wait_scatter(u2)
            issue_idx(lax.rem(b + 2, nblk), u2)
            wait_idx(u1)
            issue_gather(u1)
            wait_gather(u)
            issue_scatter(u)
            count_block(u)
            return carry

        lax.fori_loop(0, nblk, body, 0)
        for j in range(nblk + 2 - NB, nblk):
            wait_scatter(j % NB)
        wait_gather(nblk % NB)
        wait_idx((nblk + 1) % NB)
        plsc.subcore_barrier()

        def write_slice(o, n):
            pltpu.sync_copy(acc.at[pl.ds(o, n)], out_r.at[cid, pl.ds(o, n)])
        rows_of(write_slice)
        if with_cnt:
            pltpu.sync_copy(cntloc, cnt_r.at[cid, sid])

    zero = jnp.zeros((rpt, D), jnp.float32)
    if with_cnt:
        return k(table, edge_index, zero, jnp.zeros((N,), jnp.float32))
    return k(table, edge_index, zero)


def _ln_relu(h, g, b):
    mu = jnp.mean(h, axis=-1, keepdims=True)
    var = jnp.mean((h - mu) ** 2, axis=-1, keepdims=True)
    h = (h - mu) * lax.rsqrt(var + 1e-5) * g + b
    return jnp.maximum(h, 0.0)


def _tc_call(body, n_out, N, R, in_specs_dims, out_dims, *args):
    grid = (N // R,)

    def spec(dims):
        blk = tuple(R if d is None else d for d in dims)
        rowpos = [j for j, d in enumerate(dims) if d is None]
        if rowpos:
            p = rowpos[0]
            return pl.BlockSpec(blk, lambda i, p=p: tuple(
                i if j == p else 0 for j in range(len(blk))))
        return pl.BlockSpec(blk, lambda i: (0,) * len(blk))

    return pl.pallas_call(
        body,
        grid=grid,
        in_specs=[spec(d) for d in in_specs_dims],
        out_specs=[spec(d) for d in out_dims],
        out_shape=[jax.ShapeDtypeStruct(
            tuple(N if d is None else d for d in dims), jnp.float32)
            for dims in out_dims],
    )(*args)


def _dot(a, b):
    return jnp.dot(a, b, preferred_element_type=jnp.float32)


def kernel(z, edge_index, W1l, W1r, b1, W2l, W2r, b2, W3l, W3r, b3,
           W4l, W4r, b4, g1, be1, g2, be2, g3, be3):
    N, D = z.shape
    R = 1000

    b1r, g1r, be1r = b1[None, :], g1[None, :], be1[None, :]
    b2r, g2r, be2r = b2[None, :], g2[None, :], be2[None, :]
    b3r, g3r, be3r = b3[None, :], g3[None, :], be3[None, :]
    W4lp = jnp.pad(W4l, ((0, 0), (0, 16 - W4l.shape[1])))
    W4rp = jnp.pad(W4r, ((0, 0), (0, 16 - W4r.shape[1])))
    b4p = jnp.pad(b4, (0, 16 - b4.shape[0]))[None, :]

    s1, cntp = _segsum_sc(z, edge_index, with_cnt=True)
    cntT = cntp.reshape(_NC * _NS, N).T

    def tc1(s_r, c_r, z_r, wl_r, wr_r, b_r, g_r, be_r, oa_r, ob_r, inv_r):
        c = jnp.maximum(jnp.sum(c_r[...], axis=1), 1.0)
        inv = (1.0 / c)[:, None]
        mean = (s_r[0] + s_r[1]) * inv
        h = _dot(mean, wl_r[...]) + _dot(z_r[...], wr_r[...]) + b_r[...]
        h = _ln_relu(h, g_r[...], be_r[...])
        oa_r[...] = h[:, :128]
        ob_r[...] = h[:, 128:]
        inv_r[...] = jnp.broadcast_to(inv, inv_r.shape)

    x1a, x1b, inv = _tc_call(
        tc1, 3, N, R,
        [(2, None, 128), (None, 32), (None, 128), (128, 256), (128, 256),
         (1, 256), (1, 256), (1, 256)],
        [(None, 128), (None, 128), (None, 16)],
        s1, cntT, z, W1l, W1r, b1r, g1r, be1r)

    sa = _segsum_sc(x1a, edge_index, with_cnt=False)[0]
    sb = _segsum_sc(x1b, edge_index, with_cnt=False)[0]

    def tc2(sa_r, sb_r, inv_r, xa_r, xb_r, wl_r, wr_r, b_r, g_r, be_r,
            w3l_r, o_r, y3_r):
        iv = inv_r[:, :1]
        ma = (sa_r[0] + sa_r[1]) * iv
        mb = (sb_r[0] + sb_r[1]) * iv
        wl = wl_r[...]
        wr = wr_r[...]
        h = (_dot(ma, wl[:128]) + _dot(mb, wl[128:])
             + _dot(xa_r[...], wr[:128]) + _dot(xb_r[...], wr[128:])
             + b_r[...])
        h = _ln_relu(h, g_r[...], be_r[...])
        o_r[...] = h
        y3_r[...] = _dot(h, w3l_r[...])

    x2, y3 = _tc_call(
        tc2, 2, N, R,
        [(2, None, 128), (2, None, 128), (None, 16), (None, 128),
         (None, 128), (256, 256), (256, 256), (1, 256), (1, 256), (1, 256),
         (256, 128)],
        [(None, 256), (None, 128)],
        sa, sb, inv, x1a, x1b, W2l, W2r, b2r, g2r, be2r, W3l)

    s3 = _segsum_sc(y3, edge_index, with_cnt=False)[0]

    def tc3(s_r, inv_r, x2_r, wr_r, b_r, g_r, be_r, w4l_r, o_r, y4_r):
        mean = (s_r[0] + s_r[1]) * inv_r[:, :1]
        h = mean + _dot(x2_r[...], wr_r[...]) + b_r[...]
        h = _ln_relu(h, g_r[...], be_r[...])
        o_r[...] = h
        y4_r[...] = _dot(h, w4l_r[...])

    x3, y4 = _tc_call(
        tc3, 2, N, R,
        [(2, None, 128), (None, 16), (None, 256), (256, 128), (1, 128),
         (1, 128), (1, 128), (128, 16)],
        [(None, 128), (None, 16)],
        s3, inv, x2, W3r, b3r, g3r, be3r, W4lp)

    s4 = _segsum_sc(y4, edge_index, with_cnt=False)[0]

    def tc4(s_r, inv_r, x3_r, wr_r, b_r, o_r):
        mean = (s_r[0] + s_r[1]) * inv_r[:, :1]
        o_r[...] = mean + _dot(x3_r[...], wr_r[...]) + b_r[...]

    (o,) = _tc_call(
        tc4, 1, N, R,
        [(2, None, 16), (None, 16), (None, 128), (128, 16), (1, 16)],
        [(None, 16)],
        s4, inv, x3, W4rp, b4p)

    return o[:, :9]
